# baseline (speedup 1.0000x reference)
import functools

import numpy as np
import jax
import jax.numpy as jnp
from jax import lax
from jax.experimental import pallas as pl
from jax.experimental.pallas import tpu as pltpu
from jax.experimental.pallas import tpu_sc as plsc

_NC, _NS = 2, 16
_NW = _NC * _NS

_CHUNK = 256
_NBUF = 4

_BB = 128


def _sc_gather(table, idx_flat):
    n = idx_flat.shape[0]
    v_rows, d = table.shape
    per_w = n // _NW
    nchunks = per_w // _CHUNK
    nloops = nchunks // _NBUF
    assert per_w * _NW == n and _CHUNK * nchunks == per_w and _NBUF * nloops == nchunks

    mesh = plsc.VectorSubcoreMesh(
        core_axis_name="c", subcore_axis_name="s", num_cores=_NC, num_subcores=_NS
    )

    @functools.partial(
        pl.kernel,
        out_type=jax.ShapeDtypeStruct((n, d), jnp.float32),
        mesh=mesh,
        scratch_types=[
            pltpu.VMEM((per_w,), jnp.int32),
            pltpu.VMEM((per_w,), jnp.int32),
            *[pltpu.VMEM((_CHUNK, d), jnp.float32) for _ in range(_NBUF)],
            *[pltpu.SemaphoreType.DMA for _ in range(2 * _NBUF)],
        ],
        compiler_params=pltpu.CompilerParams(
            use_tc_tiling_on_sc=False, needs_layout_passes=False
        ),
    )
    def gather_kernel(table_hbm, idx_hbm, out_hbm, idx_v, idx_p, *rest):
        rows = rest[:_NBUF]
        gsems = rest[_NBUF : 2 * _NBUF]
        wsems = rest[2 * _NBUF :]
        wid = lax.axis_index("s") * _NC + lax.axis_index("c")
        base = wid * per_w
        pltpu.sync_copy(idx_hbm.at[pl.ds(base, per_w)], idx_v)

        half = per_w // 2
        hv = v_rows // 2

        def remap(v):
            return jnp.where(v >= hv, 2 * (v - hv) + 1, 2 * v)

        def perm_body(k, carry):
            va = idx_v[pl.ds(k * 16, 16)]
            vb = idx_v[pl.ds(half + k * 16, 16)]
            lanes = k * 32 + lax.broadcasted_iota(jnp.int32, (16,), 0) * 2
            plsc.store_scatter(idx_p, [lanes], remap(va))
            plsc.store_scatter(idx_p, [lanes + 1], remap(vb))
            return carry

        lax.fori_loop(0, half // 16, perm_body, 0)

        def g_start(c, b):
            pltpu.async_copy(
                table_hbm.at[idx_p.at[pl.ds(c * _CHUNK, _CHUNK)]], rows[b], gsems[b]
            )

        def g_wait(b):
            pltpu.make_async_copy(
                table_hbm.at[idx_p.at[pl.ds(0, _CHUNK)]], rows[b], gsems[b]
            ).wait()

        def w_start(c, b):
            pltpu.async_copy(
                rows[b], out_hbm.at[pl.ds(base + c * _CHUNK, _CHUNK)], wsems[b]
            )

        def w_wait(b):
            pltpu.make_async_copy(
                rows[b], out_hbm.at[pl.ds(0, _CHUNK)], wsems[b]
            ).wait()

        for b in range(_NBUF):
            g_start(b, b)

        def body(i, carry):
            for b in range(_NBUF):
                g_wait(b)
                w_start(i * _NBUF + b, b)

            @pl.when(i < nloops - 1)
            def _prefetch():
                for b in range(_NBUF):
                    w_wait(b)
                    g_start(i * _NBUF + b + _NBUF, b)

            return carry

        lax.fori_loop(0, nloops, body, 0)
        for b in range(_NBUF):
            w_wait(b)

    return gather_kernel(table, idx_flat)


def _prepack_body(va_ref, vb_ref, w1a_ref, o_ref):
    pa = jnp.dot(va_ref[...], w1a_ref[...], preferred_element_type=jnp.float32)
    pb = jnp.dot(vb_ref[...], w1a_ref[...], preferred_element_type=jnp.float32)
    o_ref[...] = jnp.concatenate([pa, pb], axis=1)


def _prepack(v2e, w1a):
    v, d = v2e.shape
    rb = min(2000, v // 2)
    nb = v // 2 // rb
    assert v // 2 % rb == 0
    return pl.pallas_call(
        _prepack_body,
        grid=(nb,),
        in_specs=[
            pl.BlockSpec((rb, d), lambda i: (i, 0)),
            pl.BlockSpec((rb, d), lambda i: (i + nb, 0)),
            pl.BlockSpec((d, d), lambda i: (0, 0)),
        ],
        out_specs=pl.BlockSpec((rb, 2 * d), lambda i: (i, 0)),
        out_shape=jax.ShapeDtypeStruct((v // 2, 2 * d), jnp.float32),
    )(v2e, v2e, w1a)


def _tc_mlp_body(
    g_ref, rt_ref, r2e_ref, w1b_ref, b1_ref,
    w2blk_ref, b2p_ref, out_ref, *, bb, ll, d, r
):
    bbl = bb * ll
    bbl2 = bbl // 2
    rtab = (
        jnp.dot(r2e_ref[...], w1b_ref[...], preferred_element_type=jnp.float32)
        + b1_ref[...]
    )
    zeros = jnp.zeros((8, d), jnp.float32)
    rtab16 = jnp.concatenate(
        [
            jnp.concatenate([rtab, zeros], axis=1),
            jnp.concatenate([zeros, rtab], axis=1),
        ],
        axis=0,
    )
    rt_row = rt_ref[0]
    iot = lax.broadcasted_iota(jnp.int32, (8, bbl2), 0)
    oh16 = jnp.concatenate(
        [
            jnp.equal(iot, rt_row[:, :bbl2]).astype(jnp.float32),
            jnp.equal(iot, rt_row[:, bbl2:]).astype(jnp.float32),
        ],
        axis=0,
    )
    radd = lax.dot_general(
        oh16, rtab16, (((0,), (0,)), ((), ())), preferred_element_type=jnp.float32
    )
    x = g_ref[...]
    h = jnp.maximum(x + radd, 0.0)
    o = jnp.maximum(
        jnp.dot(h, w2blk_ref[...], preferred_element_type=jnp.float32) + b2p_ref[...],
        0.0,
    )
    s = o.reshape(bb // 2, ll, 2 * d).sum(axis=1)
    out_ref[...] = (
        jnp.concatenate([s[:, :d], s[:, d:]], axis=0) * (1.0 / ll)
    )


def _tc_mlp(g2, rt, r2e8, w1b, b1, w2blk, b2p, *, b_total, ll, d, r):
    nblocks = b_total // _BB
    bbl = _BB * ll
    body = functools.partial(_tc_mlp_body, bb=_BB, ll=ll, d=d, r=r)
    return pl.pallas_call(
        body,
        grid=(nblocks,),
        in_specs=[
            pl.BlockSpec((bbl // 2, 2 * d), lambda i: (i, 0)),
            pl.BlockSpec((1, 1, bbl), lambda i: (i, 0, 0)),
            pl.BlockSpec((8, d), lambda i: (0, 0)),
            pl.BlockSpec((d, d), lambda i: (0, 0)),
            pl.BlockSpec((1, d), lambda i: (0, 0)),
            pl.BlockSpec((2 * d, 2 * d), lambda i: (0, 0)),
            pl.BlockSpec((1, 2 * d), lambda i: (0, 0)),
        ],
        out_specs=pl.BlockSpec((_BB, d), lambda i: (i, 0)),
        out_shape=jax.ShapeDtypeStruct((b_total, d), jnp.float32),
    )(g2, rt, r2e8, w1b, b1, w2blk, b2p)


def kernel(nodes, history_uv, history_r, v2e_weight, u2e_weight, r2e_weight, W1, b1, W2, b2):
    b_total, ll = history_uv.shape
    _, d = v2e_weight.shape
    r = r2e_weight.shape[0]
    n = b_total * ll

    w1a = W1[:d, :]
    w1b = W1[d:, :]
    t1 = _prepack(v2e_weight, w1a).reshape(-1, d)

    idx_flat = history_uv.reshape(-1).astype(jnp.int32)
    g_flat = _sc_gather(t1, idx_flat)
    g2 = g_flat.reshape(n // 2, 2 * d)

    rt = history_r.astype(jnp.int32).reshape(b_total // _BB, 1, _BB * ll)

    r2e8 = jnp.pad(r2e_weight, ((0, 8 - r), (0, 0)))
    zero = jnp.zeros((d, d), jnp.float32)
    w2blk = jnp.block([[W2, zero], [zero, W2]])
    b2p = jnp.concatenate([b2, b2]).reshape(1, 2 * d)

    return _tc_mlp(
        g2, rt, r2e8, w1b, b1.reshape(1, d),
        w2blk, b2p, b_total=b_total, ll=ll, d=d, r=r,
    )


# speedup vs baseline: 1.1463x; 1.1463x over previous
import functools

import numpy as np
import jax
import jax.numpy as jnp
from jax import lax
from jax.experimental import pallas as pl
from jax.experimental.pallas import tpu as pltpu
from jax.experimental.pallas import tpu_sc as plsc

_NC, _NS = 2, 16
_NW = _NC * _NS

_CHUNK = 256
_NBUF = 5

_BB = 128
_NSTAGE = 4


def _prepack_body(vta_ref, vtb_ref, w1a_ref, o_ref):
    dims = (((0,), (0,)), ((), ()))
    pa = lax.dot_general(
        vta_ref[...], w1a_ref[...], dims, preferred_element_type=jnp.float32
    )
    pb = lax.dot_general(
        vtb_ref[...], w1a_ref[...], dims, preferred_element_type=jnp.float32
    )
    o_ref[...] = jnp.concatenate([pa, pb], axis=1)


def _pad_items(v):
    rb = 2048
    hv = -(-v // (2 * rb)) * rb
    return rb, hv


def _prepack(v2e_t, w1a):
    d, v = v2e_t.shape
    rb, hv = _pad_items(v)
    nb = hv // rb
    nb_in = -(-v // rb)

    def b_map(i):
        return (0, jnp.minimum(i + nb, nb_in - 1))

    return pl.pallas_call(
        _prepack_body,
        grid=(nb,),
        in_specs=[
            pl.BlockSpec((d, rb), lambda i: (0, i)),
            pl.BlockSpec((d, rb), b_map),
            pl.BlockSpec((d, d), lambda i: (0, 0)),
        ],
        out_specs=pl.BlockSpec((rb, 2 * d), lambda i: (i, 0)),
        out_shape=jax.ShapeDtypeStruct((hv, 2 * d), jnp.float32),
    )(v2e_t, v2e_t, w1a)


def _sc_gather_stage(table, idx_flat, stage, bbl):
    n = idx_flat.shape[0]
    v_rows, d = table.shape
    stage_tokens = n // _NSTAGE
    sb = stage_tokens // bbl
    wpb = _NW // sb
    per_w = stage_tokens // _NW
    half = per_w // 2
    nchunks = per_w // _CHUNK
    nloops = nchunks // _NBUF
    assert sb * bbl == stage_tokens and wpb * sb == _NW
    assert _CHUNK * nchunks == per_w and _NBUF * nloops == nchunks

    mesh = plsc.VectorSubcoreMesh(
        core_axis_name="c", subcore_axis_name="s", num_cores=_NC, num_subcores=_NS
    )

    @functools.partial(
        pl.kernel,
        out_type=jax.ShapeDtypeStruct((stage_tokens, d), jnp.float32),
        mesh=mesh,
        scratch_types=[
            pltpu.VMEM((per_w,), jnp.int32),
            pltpu.VMEM((per_w,), jnp.int32),
            *[pltpu.VMEM((_CHUNK, d), jnp.float32) for _ in range(_NBUF)],
            *[pltpu.SemaphoreType.DMA for _ in range(2 * _NBUF)],
        ],
        compiler_params=pltpu.CompilerParams(
            use_tc_tiling_on_sc=False, needs_layout_passes=False
        ),
    )
    def gather_kernel(table_hbm, idx_hbm, out_hbm, idx_v, idx_p, *rest):
        rows = rest[:_NBUF]
        gsems = rest[_NBUF : 2 * _NBUF]
        wsems = rest[2 * _NBUF :]
        wid = lax.axis_index("s") * _NC + lax.axis_index("c")
        ib = wid // wpb
        q = wid % wpb
        base_a = (stage * sb + ib) * bbl + q * half
        base_b = base_a + bbl // 2
        base_out = wid * per_w
        pltpu.sync_copy(idx_hbm.at[pl.ds(base_a, half)], idx_v.at[pl.ds(0, half)])
        pltpu.sync_copy(idx_hbm.at[pl.ds(base_b, half)], idx_v.at[pl.ds(half, half)])

        hv = v_rows // 2

        def remap(v):
            return jnp.where(v >= hv, 2 * (v - hv) + 1, 2 * v)

        def perm_body(k, carry):
            va = idx_v[pl.ds(k * 16, 16)]
            vb = idx_v[pl.ds(half + k * 16, 16)]
            lanes = k * 32 + lax.broadcasted_iota(jnp.int32, (16,), 0) * 2
            plsc.store_scatter(idx_p, [lanes], remap(va))
            plsc.store_scatter(idx_p, [lanes + 1], remap(vb))
            return carry

        lax.fori_loop(0, half // 16, perm_body, 0)

        def g_start(c, b):
            pltpu.async_copy(
                table_hbm.at[idx_p.at[pl.ds(c * _CHUNK, _CHUNK)]], rows[b], gsems[b]
            )

        def g_wait(b):
            pltpu.make_async_copy(
                table_hbm.at[idx_p.at[pl.ds(0, _CHUNK)]], rows[b], gsems[b]
            ).wait()

        def w_start(c, b):
            pltpu.async_copy(
                rows[b], out_hbm.at[pl.ds(base_out + c * _CHUNK, _CHUNK)], wsems[b]
            )

        def w_wait(b):
            pltpu.make_async_copy(
                rows[b], out_hbm.at[pl.ds(0, _CHUNK)], wsems[b]
            ).wait()

        for b in range(_NBUF):
            g_start(b, b)

        def body(i, carry):
            for b in range(_NBUF):
                g_wait(b)
                w_start(i * _NBUF + b, b)

            @pl.when(i < nloops - 1)
            def _prefetch():
                for b in range(_NBUF):
                    w_wait(b)
                    g_start(i * _NBUF + b + _NBUF, b)

            return carry

        lax.fori_loop(0, nloops, body, 0)
        for b in range(_NBUF):
            w_wait(b)

    return gather_kernel(table, idx_flat)


def _tc_mlp_body(
    g_ref, rt_ref, r2e_ref, w1b_ref, b1_ref,
    w2blk_ref, b2p_ref, out_ref, *, bb, ll, d, r
):
    bbl = bb * ll
    bbl2 = bbl // 2
    rtab = (
        jnp.dot(r2e_ref[...], w1b_ref[...], preferred_element_type=jnp.float32)
        + b1_ref[...]
    )
    zeros = jnp.zeros((8, d), jnp.float32)
    rtab16 = jnp.concatenate(
        [
            jnp.concatenate([rtab, zeros], axis=1),
            jnp.concatenate([zeros, rtab], axis=1),
        ],
        axis=0,
    )
    rt_row = rt_ref[0]
    iot = lax.broadcasted_iota(jnp.int32, (8, bbl2), 0)
    oh16 = jnp.concatenate(
        [
            jnp.equal(iot, rt_row[:, :bbl2]).astype(jnp.float32),
            jnp.equal(iot, rt_row[:, bbl2:]).astype(jnp.float32),
        ],
        axis=0,
    )
    radd = lax.dot_general(
        oh16, rtab16, (((0,), (0,)), ((), ())), preferred_element_type=jnp.float32
    )
    x = g_ref[...]
    h = jnp.maximum(x + radd, 0.0)
    o = jnp.maximum(
        jnp.dot(h, w2blk_ref[...], preferred_element_type=jnp.float32) + b2p_ref[...],
        0.0,
    )
    s = o.reshape(bb // 2, ll, 2 * d).sum(axis=1)
    out_ref[...] = (
        jnp.concatenate([s[:, :d], s[:, d:]], axis=0) * (1.0 / ll)
    )


def _tc_mlp_stage(g2, rt, r2e8, w1b, b1, w2blk, b2p, stage, *, sb, ll, d, r):
    bbl = _BB * ll
    body = functools.partial(_tc_mlp_body, bb=_BB, ll=ll, d=d, r=r)
    return pl.pallas_call(
        body,
        grid=(sb,),
        in_specs=[
            pl.BlockSpec((bbl // 2, 2 * d), lambda i: (i, 0)),
            pl.BlockSpec((1, 1, bbl), lambda i: (i + stage * sb, 0, 0)),
            pl.BlockSpec((8, d), lambda i: (0, 0)),
            pl.BlockSpec((d, d), lambda i: (0, 0)),
            pl.BlockSpec((1, d), lambda i: (0, 0)),
            pl.BlockSpec((2 * d, 2 * d), lambda i: (0, 0)),
            pl.BlockSpec((1, 2 * d), lambda i: (0, 0)),
        ],
        out_specs=pl.BlockSpec((_BB, d), lambda i: (i, 0)),
        out_shape=jax.ShapeDtypeStruct((sb * _BB, d), jnp.float32),
    )(g2, rt, r2e8, w1b, b1, w2blk, b2p)


def kernel(nodes, history_uv, history_r, v2e_weight, u2e_weight, r2e_weight, W1, b1, W2, b2):
    b_total, ll = history_uv.shape
    _, d = v2e_weight.shape
    r = r2e_weight.shape[0]
    n = b_total * ll
    bbl = _BB * ll
    sb = b_total // _BB // _NSTAGE

    w1a = W1[:d, :]
    w1b = W1[d:, :]
    t1 = _prepack(v2e_weight.T, w1a).reshape(-1, d)

    idx_flat = history_uv.reshape(-1).astype(jnp.int32)
    rt = history_r.astype(jnp.int32).reshape(b_total // _BB, 1, bbl)

    r2e8 = jnp.pad(r2e_weight, ((0, 8 - r), (0, 0)))
    zero = jnp.zeros((d, d), jnp.float32)
    w2blk = jnp.block([[W2, zero], [zero, W2]])
    b2p = jnp.concatenate([b2, b2]).reshape(1, 2 * d)
    b1r = b1.reshape(1, d)

    outs = []
    for stage in range(_NSTAGE):
        g = _sc_gather_stage(t1, idx_flat, stage, bbl)
        g2 = g.reshape(-1, 2 * d)
        outs.append(
            _tc_mlp_stage(
                g2, rt, r2e8, w1b, b1r, w2blk, b2p, stage, sb=sb, ll=ll, d=d, r=r
            )
        )
    return jnp.concatenate(outs, axis=0)


# speedup vs baseline: 1.1626x; 1.0142x over previous
import functools

import numpy as np
import jax
import jax.numpy as jnp
from jax import lax
from jax.experimental import pallas as pl
from jax.experimental.pallas import tpu as pltpu
from jax.experimental.pallas import tpu_sc as plsc

_NC, _NS = 2, 16
_NW = _NC * _NS

_CHUNK = 256
_NBUF = 5

_BB = 128
_NSTAGE = 4


def _prepack_body(vta_ref, vtb_ref, w1a_ref, o_ref):
    dims = (((0,), (0,)), ((), ()))
    pa = lax.dot_general(
        vta_ref[...], w1a_ref[...], dims, preferred_element_type=jnp.float32
    )
    pb = lax.dot_general(
        vtb_ref[...], w1a_ref[...], dims, preferred_element_type=jnp.float32
    )
    o_ref[...] = jnp.concatenate([pa, pb], axis=1)


def _pad_items(v):
    rb = 4096
    hv = -(-v // (2 * rb)) * rb
    return rb, hv


def _prepack(v2e_t, w1a):
    d, v = v2e_t.shape
    rb, hv = _pad_items(v)
    nb = hv // rb
    nb_in = -(-v // rb)

    def b_map(i):
        return (0, jnp.minimum(i + nb, nb_in - 1))

    return pl.pallas_call(
        _prepack_body,
        grid=(nb,),
        in_specs=[
            pl.BlockSpec((d, rb), lambda i: (0, i)),
            pl.BlockSpec((d, rb), b_map),
            pl.BlockSpec((d, d), lambda i: (0, 0)),
        ],
        out_specs=pl.BlockSpec((rb, 2 * d), lambda i: (i, 0)),
        out_shape=jax.ShapeDtypeStruct((hv, 2 * d), jnp.float32),
    )(v2e_t, v2e_t, w1a)


def _sc_gather_stage(table, idx_flat, stage, bbl):
    n = idx_flat.shape[0]
    v_rows, d = table.shape
    stage_tokens = n // _NSTAGE
    sb = stage_tokens // bbl
    wpb = _NW // sb
    per_w = stage_tokens // _NW
    half = per_w // 2
    nchunks = per_w // _CHUNK
    nloops = nchunks // _NBUF
    assert sb * bbl == stage_tokens and wpb * sb == _NW
    assert _CHUNK * nchunks == per_w and _NBUF * nloops == nchunks

    mesh = plsc.VectorSubcoreMesh(
        core_axis_name="c", subcore_axis_name="s", num_cores=_NC, num_subcores=_NS
    )

    @functools.partial(
        pl.kernel,
        out_type=jax.ShapeDtypeStruct((stage_tokens, d), jnp.float32),
        mesh=mesh,
        scratch_types=[
            pltpu.VMEM((per_w,), jnp.int32),
            pltpu.VMEM((per_w,), jnp.int32),
            *[pltpu.VMEM((_CHUNK, d), jnp.float32) for _ in range(_NBUF)],
            *[pltpu.SemaphoreType.DMA for _ in range(2 * _NBUF)],
        ],
        compiler_params=pltpu.CompilerParams(
            use_tc_tiling_on_sc=False, needs_layout_passes=False
        ),
    )
    def gather_kernel(table_hbm, idx_hbm, out_hbm, idx_v, idx_p, *rest):
        rows = rest[:_NBUF]
        gsems = rest[_NBUF : 2 * _NBUF]
        wsems = rest[2 * _NBUF :]
        wid = lax.axis_index("s") * _NC + lax.axis_index("c")
        ib = wid // wpb
        q = wid % wpb
        base_a = (stage * sb + ib) * bbl + q * half
        base_b = base_a + bbl // 2
        base_out = wid * per_w
        pltpu.sync_copy(idx_hbm.at[pl.ds(base_a, half)], idx_v.at[pl.ds(0, half)])
        pltpu.sync_copy(idx_hbm.at[pl.ds(base_b, half)], idx_v.at[pl.ds(half, half)])

        hv = v_rows // 2

        def remap(v):
            return jnp.where(v >= hv, 2 * (v - hv) + 1, 2 * v)

        def perm_body(k, carry):
            va = idx_v[pl.ds(k * 16, 16)]
            vb = idx_v[pl.ds(half + k * 16, 16)]
            lanes = k * 32 + lax.broadcasted_iota(jnp.int32, (16,), 0) * 2
            plsc.store_scatter(idx_p, [lanes], remap(va))
            plsc.store_scatter(idx_p, [lanes + 1], remap(vb))
            return carry

        lax.fori_loop(0, half // 16, perm_body, 0)

        def g_start(c, b):
            pltpu.async_copy(
                table_hbm.at[idx_p.at[pl.ds(c * _CHUNK, _CHUNK)]], rows[b], gsems[b]
            )

        def g_wait(b):
            pltpu.make_async_copy(
                table_hbm.at[idx_p.at[pl.ds(0, _CHUNK)]], rows[b], gsems[b]
            ).wait()

        def w_start(c, b):
            pltpu.async_copy(
                rows[b], out_hbm.at[pl.ds(base_out + c * _CHUNK, _CHUNK)], wsems[b]
            )

        def w_wait(b):
            pltpu.make_async_copy(
                rows[b], out_hbm.at[pl.ds(0, _CHUNK)], wsems[b]
            ).wait()

        for b in range(_NBUF):
            g_start(b, b)

        def body(i, carry):
            for b in range(_NBUF):
                g_wait(b)
                w_start(i * _NBUF + b, b)

            @pl.when(i < nloops - 1)
            def _prefetch():
                for b in range(_NBUF):
                    w_wait(b)
                    g_start(i * _NBUF + b + _NBUF, b)

            return carry

        lax.fori_loop(0, nloops, body, 0)
        for b in range(_NBUF):
            w_wait(b)

    return gather_kernel(table, idx_flat)


def _tc_mlp_body(
    g_ref, rt_ref, r2e_ref, w1b_ref, b1_ref,
    w2blk_ref, b2p_ref, out_ref, *, bb, ll, d, r
):
    bbl = bb * ll
    bbl2 = bbl // 2
    rtab = (
        jnp.dot(r2e_ref[...], w1b_ref[...], preferred_element_type=jnp.float32)
        + b1_ref[...]
    )
    zeros = jnp.zeros((8, d), jnp.float32)
    rtab16 = jnp.concatenate(
        [
            jnp.concatenate([rtab, zeros], axis=1),
            jnp.concatenate([zeros, rtab], axis=1),
        ],
        axis=0,
    )
    rt_row = rt_ref[0]
    iot = lax.broadcasted_iota(jnp.int32, (8, bbl2), 0)
    oh16 = jnp.concatenate(
        [
            jnp.equal(iot, rt_row[:, :bbl2]).astype(jnp.float32),
            jnp.equal(iot, rt_row[:, bbl2:]).astype(jnp.float32),
        ],
        axis=0,
    )
    radd = lax.dot_general(
        oh16, rtab16, (((0,), (0,)), ((), ())), preferred_element_type=jnp.float32
    )
    x = g_ref[...]
    h = jnp.maximum(x + radd, 0.0)
    o = jnp.maximum(
        jnp.dot(h, w2blk_ref[...], preferred_element_type=jnp.float32) + b2p_ref[...],
        0.0,
    )
    s = o.reshape(bb // 2, ll, 2 * d).sum(axis=1)
    out_ref[...] = (
        jnp.concatenate([s[:, :d], s[:, d:]], axis=0) * (1.0 / ll)
    )


def _tc_mlp_stage(g2, rt, r2e8, w1b, b1, w2blk, b2p, stage, *, sb, ll, d, r):
    bbl = _BB * ll
    body = functools.partial(_tc_mlp_body, bb=_BB, ll=ll, d=d, r=r)
    return pl.pallas_call(
        body,
        grid=(sb,),
        in_specs=[
            pl.BlockSpec((bbl // 2, 2 * d), lambda i: (i, 0)),
            pl.BlockSpec((1, 1, bbl), lambda i: (i + stage * sb, 0, 0)),
            pl.BlockSpec((8, d), lambda i: (0, 0)),
            pl.BlockSpec((d, d), lambda i: (0, 0)),
            pl.BlockSpec((1, d), lambda i: (0, 0)),
            pl.BlockSpec((2 * d, 2 * d), lambda i: (0, 0)),
            pl.BlockSpec((1, 2 * d), lambda i: (0, 0)),
        ],
        out_specs=pl.BlockSpec((_BB, d), lambda i: (i, 0)),
        out_shape=jax.ShapeDtypeStruct((sb * _BB, d), jnp.float32),
    )(g2, rt, r2e8, w1b, b1, w2blk, b2p)


def kernel(nodes, history_uv, history_r, v2e_weight, u2e_weight, r2e_weight, W1, b1, W2, b2):
    b_total, ll = history_uv.shape
    _, d = v2e_weight.shape
    r = r2e_weight.shape[0]
    n = b_total * ll
    bbl = _BB * ll
    sb = b_total // _BB // _NSTAGE

    w1a = W1[:d, :]
    w1b = W1[d:, :]
    t1 = _prepack(v2e_weight.T, w1a).reshape(-1, d)

    idx_flat = history_uv.reshape(-1).astype(jnp.int32)
    rt = history_r.astype(jnp.int32).reshape(b_total // _BB, 1, bbl)

    r2e8 = jnp.pad(r2e_weight, ((0, 8 - r), (0, 0)))
    zero = jnp.zeros((d, d), jnp.float32)
    w2blk = jnp.block([[W2, zero], [zero, W2]])
    b2p = jnp.concatenate([b2, b2]).reshape(1, 2 * d)
    b1r = b1.reshape(1, d)

    outs = []
    for stage in range(_NSTAGE):
        g = _sc_gather_stage(t1, idx_flat, stage, bbl)
        g2 = g.reshape(-1, 2 * d)
        outs.append(
            _tc_mlp_stage(
                g2, rt, r2e8, w1b, b1r, w2blk, b2p, stage, sb=sb, ll=ll, d=d, r=r
            )
        )
    return jnp.concatenate(outs, axis=0)


# speedup vs baseline: 1.5037x; 1.2933x over previous
import functools

import numpy as np
import jax
import jax.numpy as jnp
from jax import lax
from jax.experimental import pallas as pl
from jax.experimental.pallas import tpu as pltpu
from jax.experimental.pallas import tpu_sc as plsc

_NC, _NS = 2, 16
_NW = _NC * _NS

_CHUNK = 256
_NBUF = 5

_BB = 128
_NSTAGE = 4

_RB = 4096


def _pack_bf16(p):
    d2 = p.shape[1] // 2
    u = lax.bitcast_convert_type(p, jnp.uint32)
    rnd = jnp.uint32(0x8000)
    hi = (u[:, :d2] + rnd) & jnp.uint32(0xFFFF0000)
    lo = (u[:, d2:] + rnd) >> 16
    return lax.bitcast_convert_type(hi | lo, jnp.float32)


def _prepack_body(va_ref, vb_ref, vc_ref, vd_ref, w1a_ref, o_ref):
    dims = (((0,), (0,)), ((), ()))
    packs = []
    for ref in (va_ref, vb_ref, vc_ref, vd_ref):
        p = lax.dot_general(
            ref[...], w1a_ref[...], dims, preferred_element_type=jnp.float32
        )
        packs.append(_pack_bf16(p))
    o_ref[...] = jnp.concatenate(packs, axis=1)


def _quad_pad(v):
    hv4 = -(-v // (4 * _RB)) * _RB
    return hv4


def _prepack(v2e_t, w1a):
    d, v = v2e_t.shape
    hv4 = _quad_pad(v)
    nb = hv4 // _RB
    nb_in = -(-v // _RB)

    def strip_map(k):
        return lambda i: (0, jnp.minimum(i + k * nb, nb_in - 1))

    return pl.pallas_call(
        _prepack_body,
        grid=(nb,),
        in_specs=[
            pl.BlockSpec((d, _RB), strip_map(0)),
            pl.BlockSpec((d, _RB), strip_map(1)),
            pl.BlockSpec((d, _RB), strip_map(2)),
            pl.BlockSpec((d, _RB), strip_map(3)),
            pl.BlockSpec((d, d), lambda i: (0, 0)),
        ],
        out_specs=pl.BlockSpec((_RB, 2 * d), lambda i: (i, 0)),
        out_shape=jax.ShapeDtypeStruct((hv4, 2 * d), jnp.float32),
    )(v2e_t, v2e_t, v2e_t, v2e_t, w1a)


def _sc_gather_stage(table, idx_flat, stage, bbl):
    n = idx_flat.shape[0]
    vp, dw = table.shape
    hv4 = vp // 4
    stage_tokens = n // _NSTAGE
    sb = stage_tokens // bbl
    wpb = _NW // sb
    per_w = stage_tokens // _NW
    qlen = per_w // 4
    nchunks = per_w // _CHUNK
    nloops = nchunks // _NBUF
    assert sb * bbl == stage_tokens and wpb * sb == _NW
    assert _CHUNK * nchunks == per_w and _NBUF * nloops == nchunks and qlen % 16 == 0

    mesh = plsc.VectorSubcoreMesh(
        core_axis_name="c", subcore_axis_name="s", num_cores=_NC, num_subcores=_NS
    )

    @functools.partial(
        pl.kernel,
        out_type=jax.ShapeDtypeStruct((stage_tokens, dw), jnp.float32),
        mesh=mesh,
        scratch_types=[
            pltpu.VMEM((per_w,), jnp.int32),
            pltpu.VMEM((per_w,), jnp.int32),
            *[pltpu.VMEM((_CHUNK, dw), jnp.float32) for _ in range(_NBUF)],
            *[pltpu.SemaphoreType.DMA for _ in range(2 * _NBUF)],
        ],
        compiler_params=pltpu.CompilerParams(
            use_tc_tiling_on_sc=False, needs_layout_passes=False
        ),
    )
    def gather_kernel(table_hbm, idx_hbm, out_hbm, idx_v, idx_p, *rest):
        rows = rest[:_NBUF]
        gsems = rest[_NBUF : 2 * _NBUF]
        wsems = rest[2 * _NBUF :]
        wid = lax.axis_index("s") * _NC + lax.axis_index("c")
        ib = wid // wpb
        q = wid % wpb
        block_base = (stage * sb + ib) * bbl
        base_out = wid * per_w
        for k in range(4):
            pltpu.sync_copy(
                idx_hbm.at[pl.ds(block_base + k * (bbl // 4) + q * qlen, qlen)],
                idx_v.at[pl.ds(k * qlen, qlen)],
            )

        def remap(v):
            m0 = 4 * v
            m1 = 4 * (v - hv4) + 1
            m2 = 4 * (v - 2 * hv4) + 2
            m3 = 4 * (v - 3 * hv4) + 3
            return jnp.where(
                v < 2 * hv4,
                jnp.where(v < hv4, m0, m1),
                jnp.where(v < 3 * hv4, m2, m3),
            )

        iota = lax.broadcasted_iota(jnp.int32, (16,), 0)

        def perm_body(kk, carry):
            for k in range(4):
                v = idx_v[pl.ds(k * qlen + kk * 16, 16)]
                lanes = kk * 64 + iota * 4 + k
                plsc.store_scatter(idx_p, [lanes], remap(v))
            return carry

        lax.fori_loop(0, qlen // 16, perm_body, 0)

        def g_start(c, b):
            pltpu.async_copy(
                table_hbm.at[idx_p.at[pl.ds(c * _CHUNK, _CHUNK)]], rows[b], gsems[b]
            )

        def g_wait(b):
            pltpu.make_async_copy(
                table_hbm.at[idx_p.at[pl.ds(0, _CHUNK)]], rows[b], gsems[b]
            ).wait()

        def w_start(c, b):
            pltpu.async_copy(
                rows[b], out_hbm.at[pl.ds(base_out + c * _CHUNK, _CHUNK)], wsems[b]
            )

        def w_wait(b):
            pltpu.make_async_copy(
                rows[b], out_hbm.at[pl.ds(0, _CHUNK)], wsems[b]
            ).wait()

        for b in range(_NBUF):
            g_start(b, b)

        def body(i, carry):
            for b in range(_NBUF):
                g_wait(b)
                w_start(i * _NBUF + b, b)

            @pl.when(i < nloops - 1)
            def _prefetch():
                for b in range(_NBUF):
                    w_wait(b)
                    g_start(i * _NBUF + b + _NBUF, b)

            return carry

        lax.fori_loop(0, nloops, body, 0)
        for b in range(_NBUF):
            w_wait(b)

    return gather_kernel(table, idx_flat)


def _tc_mlp_body(
    g_ref, rt_ref, r2e_ref, w1b_ref, b1_ref,
    aff_ref, asf_ref, afs_ref, ass_ref, b2f_ref, b2s_ref, out_ref,
    *, bb, ll, d,
):
    bbl = bb * ll
    bbl4 = bbl // 4
    d2 = d // 2
    rtab = (
        jnp.dot(r2e_ref[...], w1b_ref[...], preferred_element_type=jnp.float32)
        + b1_ref[...]
    )
    rt_f = rtab[:, :d2]
    rt_s = rtab[:, d2:]
    rt_row = rt_ref[0]
    iot = lax.broadcasted_iota(jnp.int32, (8, bbl4), 0)
    oh = jnp.concatenate(
        [
            jnp.equal(iot, rt_row[:, k * bbl4 : (k + 1) * bbl4]).astype(jnp.float32)
            for k in range(4)
        ],
        axis=0,
    )
    rtq_f = jnp.concatenate(
        [jnp.pad(rt_f, ((0, 0), (d2 * k, d2 * (3 - k)))) for k in range(4)], axis=0
    )
    rtq_s = jnp.concatenate(
        [jnp.pad(rt_s, ((0, 0), (d2 * k, d2 * (3 - k)))) for k in range(4)], axis=0
    )
    dims0 = (((0,), (0,)), ((), ()))
    radd_f = lax.dot_general(oh, rtq_f, dims0, preferred_element_type=jnp.float32)
    radd_s = lax.dot_general(oh, rtq_s, dims0, preferred_element_type=jnp.float32)

    u = lax.bitcast_convert_type(g_ref[...], jnp.uint32)
    x_f = lax.bitcast_convert_type(u & jnp.uint32(0xFFFF0000), jnp.float32)
    x_s = lax.bitcast_convert_type(u << 16, jnp.float32)
    h_f = jnp.maximum(x_f + radd_f, 0.0).astype(jnp.bfloat16)
    h_s = jnp.maximum(x_s + radd_s, 0.0).astype(jnp.bfloat16)
    o_f = jnp.maximum(
        jnp.dot(h_f, aff_ref[...], preferred_element_type=jnp.float32)
        + jnp.dot(h_s, asf_ref[...], preferred_element_type=jnp.float32)
        + b2f_ref[...],
        0.0,
    )
    o_s = jnp.maximum(
        jnp.dot(h_f, afs_ref[...], preferred_element_type=jnp.float32)
        + jnp.dot(h_s, ass_ref[...], preferred_element_type=jnp.float32)
        + b2s_ref[...],
        0.0,
    )
    s_f = o_f.reshape(bb // 4, ll, 2 * d).sum(axis=1)
    s_s = o_s.reshape(bb // 4, ll, 2 * d).sum(axis=1)
    out_ref[...] = jnp.concatenate(
        [
            jnp.concatenate(
                [s_f[:, d2 * k : d2 * (k + 1)], s_s[:, d2 * k : d2 * (k + 1)]],
                axis=1,
            )
            for k in range(4)
        ],
        axis=0,
    ) * (1.0 / ll)


def _tc_mlp_stage(g4, rt, r2e8, w1b, b1, ws, b2f, b2s, stage, *, sb, ll, d):
    bbl = _BB * ll
    body = functools.partial(_tc_mlp_body, bb=_BB, ll=ll, d=d)
    return pl.pallas_call(
        body,
        grid=(sb,),
        in_specs=[
            pl.BlockSpec((bbl // 4, 2 * d), lambda i: (i, 0)),
            pl.BlockSpec((1, 1, bbl), lambda i: (i + stage * sb, 0, 0)),
            pl.BlockSpec((8, d), lambda i: (0, 0)),
            pl.BlockSpec((d, d), lambda i: (0, 0)),
            pl.BlockSpec((1, d), lambda i: (0, 0)),
            pl.BlockSpec((2 * d, 2 * d), lambda i: (0, 0)),
            pl.BlockSpec((2 * d, 2 * d), lambda i: (0, 0)),
            pl.BlockSpec((2 * d, 2 * d), lambda i: (0, 0)),
            pl.BlockSpec((2 * d, 2 * d), lambda i: (0, 0)),
            pl.BlockSpec((1, 2 * d), lambda i: (0, 0)),
            pl.BlockSpec((1, 2 * d), lambda i: (0, 0)),
        ],
        out_specs=pl.BlockSpec((_BB, d), lambda i: (i, 0)),
        out_shape=jax.ShapeDtypeStruct((sb * _BB, d), jnp.float32),
    )(g4, rt, r2e8, w1b, b1, *ws, b2f, b2s)


def kernel(nodes, history_uv, history_r, v2e_weight, u2e_weight, r2e_weight, W1, b1, W2, b2):
    b_total, ll = history_uv.shape
    _, d = v2e_weight.shape
    r = r2e_weight.shape[0]
    d2 = d // 2
    bbl = _BB * ll
    sb = b_total // _BB // _NSTAGE

    w1a = W1[:d, :]
    w1b = W1[d:, :]
    t1 = _prepack(v2e_weight.T, w1a).reshape(-1, d2)

    idx_flat = history_uv.reshape(-1).astype(jnp.int32)
    rt = history_r.astype(jnp.int32).reshape(b_total // _BB, 1, bbl)

    r2e8 = jnp.pad(r2e_weight, ((0, 8 - r), (0, 0)))
    eye4 = jnp.eye(4, dtype=jnp.float32)
    ws = [
        jnp.kron(eye4, blk).astype(jnp.bfloat16)
        for blk in (W2[:d2, :d2], W2[d2:, :d2], W2[:d2, d2:], W2[d2:, d2:])
    ]
    b2f = jnp.tile(b2[:d2], 4).reshape(1, 2 * d)
    b2s = jnp.tile(b2[d2:], 4).reshape(1, 2 * d)
    b1r = b1.reshape(1, d)

    outs = []
    for stage in range(_NSTAGE):
        g = _sc_gather_stage(t1, idx_flat, stage, bbl)
        g4 = g.reshape(-1, 2 * d)
        outs.append(
            _tc_mlp_stage(
                g4, rt, r2e8, w1b, b1r, ws, b2f, b2s, stage, sb=sb, ll=ll, d=d
            )
        )
    return jnp.concatenate(outs, axis=0)


# speedup vs baseline: 1.5936x; 1.0598x over previous
import functools

import numpy as np
import jax
import jax.numpy as jnp
from jax import lax
from jax.experimental import pallas as pl
from jax.experimental.pallas import tpu as pltpu
from jax.experimental.pallas import tpu_sc as plsc

_NC, _NS = 2, 16
_NW = _NC * _NS

_CHUNK = 256
_NBUF = 5

_BB = 128
_NSTAGE = 4

_RB = 4096


def _prepack_body(va_ref, vb_ref, vc_ref, vd_ref, w1a_lo_ref, w1a_hi_ref, o_ref):
    dims = (((0,), (0,)), ((), ()))
    packs = []
    for ref in (va_ref, vb_ref, vc_ref, vd_ref):
        vt = ref[...]
        p_lo = lax.dot_general(
            vt, w1a_lo_ref[...], dims, preferred_element_type=jnp.float32
        )
        p_hi = lax.dot_general(
            vt, w1a_hi_ref[...], dims, preferred_element_type=jnp.float32
        )
        w = pltpu.pack_elementwise([p_lo, p_hi], packed_dtype=jnp.bfloat16)
        packs.append(lax.bitcast_convert_type(w, jnp.float32))
    o_ref[...] = jnp.concatenate(packs, axis=1)


def _quad_pad(v):
    hv4 = -(-v // (4 * _RB)) * _RB
    return hv4


def _prepack(v2e_t, w1a):
    d, v = v2e_t.shape
    hv4 = _quad_pad(v)
    nb = hv4 // _RB
    nb_in = -(-v // _RB)

    def strip_map(k):
        return lambda i: (0, jnp.minimum(i + k * nb, nb_in - 1))

    return pl.pallas_call(
        _prepack_body,
        grid=(nb,),
        in_specs=[
            pl.BlockSpec((d, _RB), strip_map(0)),
            pl.BlockSpec((d, _RB), strip_map(1)),
            pl.BlockSpec((d, _RB), strip_map(2)),
            pl.BlockSpec((d, _RB), strip_map(3)),
            pl.BlockSpec((d, d // 2), lambda i: (0, 0)),
            pl.BlockSpec((d, d // 2), lambda i: (0, 0)),
        ],
        out_specs=pl.BlockSpec((_RB, 2 * d), lambda i: (i, 0)),
        out_shape=jax.ShapeDtypeStruct((hv4, 2 * d), jnp.float32),
    )(v2e_t, v2e_t, v2e_t, v2e_t, w1a[:, : d // 2], w1a[:, d // 2 :])


def _sc_gather_stage(table, idx_flat, stage, bbl):
    n = idx_flat.shape[0]
    vp, dw = table.shape
    hv4 = vp // 4
    stage_tokens = n // _NSTAGE
    sb = stage_tokens // bbl
    wpb = _NW // sb
    per_w = stage_tokens // _NW
    qlen = per_w // 4
    nchunks = per_w // _CHUNK
    nloops = nchunks // _NBUF
    assert sb * bbl == stage_tokens and wpb * sb == _NW
    assert _CHUNK * nchunks == per_w and _NBUF * nloops == nchunks and qlen % 16 == 0

    mesh = plsc.VectorSubcoreMesh(
        core_axis_name="c", subcore_axis_name="s", num_cores=_NC, num_subcores=_NS
    )

    @functools.partial(
        pl.kernel,
        out_type=jax.ShapeDtypeStruct((stage_tokens, dw), jnp.float32),
        mesh=mesh,
        scratch_types=[
            pltpu.VMEM((per_w,), jnp.int32),
            pltpu.VMEM((per_w,), jnp.int32),
            *[pltpu.VMEM((_CHUNK, dw), jnp.float32) for _ in range(_NBUF)],
            *[pltpu.SemaphoreType.DMA for _ in range(2 * _NBUF)],
        ],
        compiler_params=pltpu.CompilerParams(
            use_tc_tiling_on_sc=False, needs_layout_passes=False
        ),
    )
    def gather_kernel(table_hbm, idx_hbm, out_hbm, idx_v, idx_p, *rest):
        rows = rest[:_NBUF]
        gsems = rest[_NBUF : 2 * _NBUF]
        wsems = rest[2 * _NBUF :]
        wid = lax.axis_index("s") * _NC + lax.axis_index("c")
        ib = wid // wpb
        q = wid % wpb
        block_base = (stage * sb + ib) * bbl
        base_out = wid * per_w
        for k in range(4):
            pltpu.sync_copy(
                idx_hbm.at[pl.ds(block_base + k * (bbl // 4) + q * qlen, qlen)],
                idx_v.at[pl.ds(k * qlen, qlen)],
            )

        def remap(v):
            m0 = 4 * v
            m1 = 4 * (v - hv4) + 1
            m2 = 4 * (v - 2 * hv4) + 2
            m3 = 4 * (v - 3 * hv4) + 3
            return jnp.where(
                v < 2 * hv4,
                jnp.where(v < hv4, m0, m1),
                jnp.where(v < 3 * hv4, m2, m3),
            )

        iota = lax.broadcasted_iota(jnp.int32, (16,), 0)

        def perm_body(kk, carry):
            for k in range(4):
                v = idx_v[pl.ds(k * qlen + kk * 16, 16)]
                lanes = kk * 64 + iota * 4 + k
                plsc.store_scatter(idx_p, [lanes], remap(v))
            return carry

        lax.fori_loop(0, qlen // 16, perm_body, 0)

        def g_start(c, b):
            pltpu.async_copy(
                table_hbm.at[idx_p.at[pl.ds(c * _CHUNK, _CHUNK)]], rows[b], gsems[b]
            )

        def g_wait(b):
            pltpu.make_async_copy(
                table_hbm.at[idx_p.at[pl.ds(0, _CHUNK)]], rows[b], gsems[b]
            ).wait()

        def w_start(c, b):
            pltpu.async_copy(
                rows[b], out_hbm.at[pl.ds(base_out + c * _CHUNK, _CHUNK)], wsems[b]
            )

        def w_wait(b):
            pltpu.make_async_copy(
                rows[b], out_hbm.at[pl.ds(0, _CHUNK)], wsems[b]
            ).wait()

        for b in range(_NBUF):
            g_start(b, b)

        def body(i, carry):
            for b in range(_NBUF):
                g_wait(b)
                w_start(i * _NBUF + b, b)

            @pl.when(i < nloops - 1)
            def _prefetch():
                for b in range(_NBUF):
                    w_wait(b)
                    g_start(i * _NBUF + b + _NBUF, b)

            return carry

        lax.fori_loop(0, nloops, body, 0)
        for b in range(_NBUF):
            w_wait(b)

    return gather_kernel(table, idx_flat)


def _tc_mlp_body(
    g_ref, rt_ref, r2e_ref, w1be_ref, w1bo_ref, b1e_ref, b1o_ref,
    aff_ref, asf_ref, afs_ref, ass_ref, b2f_ref, b2s_ref, out_ref,
    *, bb, ll, d,
):
    bbl = bb * ll
    bbl4 = bbl // 4
    d2 = d // 2
    rt_f = (
        jnp.dot(r2e_ref[...], w1be_ref[...], preferred_element_type=jnp.float32)
        + b1e_ref[...]
    )
    rt_s = (
        jnp.dot(r2e_ref[...], w1bo_ref[...], preferred_element_type=jnp.float32)
        + b1o_ref[...]
    )
    rt_row = rt_ref[0]
    iot = lax.broadcasted_iota(jnp.int32, (8, bbl4), 0)
    oh = jnp.concatenate(
        [
            jnp.equal(iot, rt_row[:, k * bbl4 : (k + 1) * bbl4]).astype(jnp.float32)
            for k in range(4)
        ],
        axis=0,
    )
    rtq_f = jnp.concatenate(
        [jnp.pad(rt_f, ((0, 0), (d2 * k, d2 * (3 - k)))) for k in range(4)], axis=0
    )
    rtq_s = jnp.concatenate(
        [jnp.pad(rt_s, ((0, 0), (d2 * k, d2 * (3 - k)))) for k in range(4)], axis=0
    )
    dims0 = (((0,), (0,)), ((), ()))
    radd_f = lax.dot_general(oh, rtq_f, dims0, preferred_element_type=jnp.float32)
    radd_s = lax.dot_general(oh, rtq_s, dims0, preferred_element_type=jnp.float32)

    u = lax.bitcast_convert_type(g_ref[...], jnp.uint32)
    x_f = lax.bitcast_convert_type(u << 16, jnp.float32)
    x_s = lax.bitcast_convert_type(u & jnp.uint32(0xFFFF0000), jnp.float32)
    h_f = jnp.maximum(x_f + radd_f, 0.0).astype(jnp.bfloat16)
    h_s = jnp.maximum(x_s + radd_s, 0.0).astype(jnp.bfloat16)
    o_f = jnp.maximum(
        jnp.dot(h_f, aff_ref[...], preferred_element_type=jnp.float32)
        + jnp.dot(h_s, asf_ref[...], preferred_element_type=jnp.float32)
        + b2f_ref[...],
        0.0,
    )
    o_s = jnp.maximum(
        jnp.dot(h_f, afs_ref[...], preferred_element_type=jnp.float32)
        + jnp.dot(h_s, ass_ref[...], preferred_element_type=jnp.float32)
        + b2s_ref[...],
        0.0,
    )
    s_f = o_f.reshape(bb // 4, ll, 2 * d).sum(axis=1)
    s_s = o_s.reshape(bb // 4, ll, 2 * d).sum(axis=1)
    out_ref[...] = jnp.concatenate(
        [
            jnp.concatenate(
                [s_f[:, d2 * k : d2 * (k + 1)], s_s[:, d2 * k : d2 * (k + 1)]],
                axis=1,
            )
            for k in range(4)
        ],
        axis=0,
    ) * (1.0 / ll)


def _tc_mlp_stage(g4, rt, r2e8, w1bs, b1s, ws, b2f, b2s, stage, *, sb, ll, d):
    bbl = _BB * ll
    d2 = d // 2
    body = functools.partial(_tc_mlp_body, bb=_BB, ll=ll, d=d)
    return pl.pallas_call(
        body,
        grid=(sb,),
        in_specs=[
            pl.BlockSpec((bbl // 4, 2 * d), lambda i: (i, 0)),
            pl.BlockSpec((1, 1, bbl), lambda i: (i + stage * sb, 0, 0)),
            pl.BlockSpec((8, d), lambda i: (0, 0)),
            pl.BlockSpec((d, d2), lambda i: (0, 0)),
            pl.BlockSpec((d, d2), lambda i: (0, 0)),
            pl.BlockSpec((1, d2), lambda i: (0, 0)),
            pl.BlockSpec((1, d2), lambda i: (0, 0)),
            pl.BlockSpec((2 * d, 2 * d), lambda i: (0, 0)),
            pl.BlockSpec((2 * d, 2 * d), lambda i: (0, 0)),
            pl.BlockSpec((2 * d, 2 * d), lambda i: (0, 0)),
            pl.BlockSpec((2 * d, 2 * d), lambda i: (0, 0)),
            pl.BlockSpec((1, 2 * d), lambda i: (0, 0)),
            pl.BlockSpec((1, 2 * d), lambda i: (0, 0)),
        ],
        out_specs=pl.BlockSpec((_BB, d), lambda i: (i, 0)),
        out_shape=jax.ShapeDtypeStruct((sb * _BB, d), jnp.float32),
    )(g4, rt, r2e8, *w1bs, *b1s, *ws, b2f, b2s)


def kernel(nodes, history_uv, history_r, v2e_weight, u2e_weight, r2e_weight, W1, b1, W2, b2):
    b_total, ll = history_uv.shape
    _, d = v2e_weight.shape
    r = r2e_weight.shape[0]
    d2 = d // 2
    bbl = _BB * ll
    sb = b_total // _BB // _NSTAGE

    w1a = W1[:d, :]
    w1b = W1[d:, :]
    t1 = _prepack(v2e_weight.T, w1a).reshape(-1, d2)

    idx_flat = history_uv.reshape(-1).astype(jnp.int32)
    rt = history_r.astype(jnp.int32).reshape(b_total // _BB, 1, bbl)

    r2e8 = jnp.pad(r2e_weight, ((0, 8 - r), (0, 0)))
    eye4 = jnp.eye(4, dtype=jnp.float32)
    w2e = W2[:d2, :]
    w2o = W2[d2:, :]
    ws = [
        jnp.kron(eye4, blk).astype(jnp.bfloat16)
        for blk in (w2e[:, :d2], w2o[:, :d2], w2e[:, d2:], w2o[:, d2:])
    ]
    w1bs = [w1b[:, :d2], w1b[:, d2:]]
    b1s = [b1[:d2].reshape(1, d2), b1[d2:].reshape(1, d2)]
    b2f = jnp.tile(b2[:d2], 4).reshape(1, 2 * d)
    b2s = jnp.tile(b2[d2:], 4).reshape(1, 2 * d)

    outs = []
    for stage in range(_NSTAGE):
        g = _sc_gather_stage(t1, idx_flat, stage, bbl)
        g4 = g.reshape(-1, 2 * d)
        outs.append(
            _tc_mlp_stage(
                g4, rt, r2e8, w1bs, b1s, ws, b2f, b2s, stage, sb=sb, ll=ll, d=d
            )
        )
    return jnp.concatenate(outs, axis=0)
